# Initial kernel scaffold; baseline (speedup 1.0000x reference)
import functools

import jax
import jax.numpy as jnp
from jax import lax
from jax.experimental import pallas as pl
from jax.experimental.pallas import tpu as pltpu
from jax.experimental.pallas import tpu_sc as plsc

N_NODES = 10000
N_EDGES = 320000
EDGE_DIM = 16
NODE_DIM = 128

NC = 2
NS = 16
NW = NC * NS
EPW = N_EDGES // NW
BATCH = 100
CHUNK = 5000
N_CHUNKS = EPW // CHUNK
BPC = CHUNK // BATCH
ROWS_PER_TILE = N_NODES // NS


def _sc_scatter_body(src2d_hbm, eattr_hbm, out_sums, out_cnts,
                     ebuf, ibuf, ones_v, zer_v, sums_sh, cnts_sh):
    c = lax.axis_index("c")
    s = lax.axis_index("s")
    w = c * NS + s

    z16 = jnp.zeros((16,), jnp.float32)
    o16 = jnp.ones((16,), jnp.float32)

    def fill_z(i, carry):
        zer_v[i, :] = z16
        return carry

    lax.fori_loop(0, ROWS_PER_TILE, fill_z, 0)

    def fill_o(i, carry):
        ones_v[i, :] = o16
        return carry

    lax.fori_loop(0, BATCH, fill_o, 0)

    pltpu.sync_copy(zer_v, sums_sh.at[pl.ds(s * ROWS_PER_TILE, ROWS_PER_TILE)])
    pltpu.sync_copy(zer_v, cnts_sh.at[pl.ds(s * ROWS_PER_TILE, ROWS_PER_TILE)])
    plsc.subcore_barrier()

    for k in range(N_CHUNKS):
        ebase = w * EPW + k * CHUNK
        irow = (w * EPW + k * CHUNK) // BATCH
        pltpu.sync_copy(eattr_hbm.at[pl.ds(ebase, CHUNK)], ebuf)
        pltpu.sync_copy(src2d_hbm.at[pl.ds(irow, BPC)], ibuf)

        def scat(b, carry):
            idx = ibuf.at[b]
            pltpu.sync_copy(ebuf.at[pl.ds(b * BATCH, BATCH)],
                            sums_sh.at[idx], add=True)
            pltpu.sync_copy(ones_v, cnts_sh.at[idx], add=True)
            return carry

        lax.fori_loop(0, BPC, scat, 0)

    plsc.subcore_barrier()

    row0 = s * ROWS_PER_TILE
    pltpu.sync_copy(sums_sh.at[pl.ds(row0, ROWS_PER_TILE)],
                    out_sums.at[c, pl.ds(row0, ROWS_PER_TILE)])
    pltpu.sync_copy(cnts_sh.at[pl.ds(row0, ROWS_PER_TILE)],
                    out_cnts.at[c, pl.ds(row0, ROWS_PER_TILE)])


@jax.jit
def _sc_scatter(src2d, edge_attr):
    mesh = plsc.VectorSubcoreMesh(core_axis_name="c", subcore_axis_name="s")
    f = pl.kernel(
        _sc_scatter_body,
        out_type=(
            jax.ShapeDtypeStruct((NC, N_NODES, EDGE_DIM), jnp.float32),
            jax.ShapeDtypeStruct((NC, N_NODES, EDGE_DIM), jnp.float32),
        ),
        mesh=mesh,
        scratch_types=[
            pltpu.VMEM((CHUNK, EDGE_DIM), jnp.float32),
            pltpu.VMEM((BPC, BATCH), jnp.int32),
            pltpu.VMEM((BATCH, EDGE_DIM), jnp.float32),
            pltpu.VMEM((ROWS_PER_TILE, EDGE_DIM), jnp.float32),
            pltpu.VMEM_SHARED((N_NODES, EDGE_DIM), jnp.float32),
            pltpu.VMEM_SHARED((N_NODES, EDGE_DIM), jnp.float32),
        ],
    )
    return f(src2d, edge_attr)


def _tc_mlp_body(x_ref, sums_ref, cnts_ref, w1a_ref, w1b_ref, b1_ref,
                 w2_ref, b2_ref, w3_ref, b3_ref, wo_ref, bo_ref,
                 g_ref, bt_ref, out_ref):
    sums = sums_ref[0] + sums_ref[1]
    cnt = cnts_ref[0, :, 0:1] + cnts_ref[1, :, 0:1]
    agg = sums / jnp.maximum(cnt, 1.0)

    g = g_ref[...]
    bt = bt_ref[...]

    h = (jnp.dot(x_ref[...], w1a_ref[...], preferred_element_type=jnp.float32)
         + jnp.dot(agg, w1b_ref[...], preferred_element_type=jnp.float32)
         + b1_ref[...])

    for w_ref, b_ref in ((w2_ref, b2_ref), (w3_ref, b3_ref), (None, None)):
        h = jnp.maximum(h, 0.0)
        mu = jnp.mean(h, axis=0, keepdims=True)
        d = h - mu
        var = jnp.mean(d * d, axis=0, keepdims=True)
        h = g * d / jnp.sqrt(var + 1e-5) + bt
        if w_ref is not None:
            h = jnp.dot(h, w_ref[...], preferred_element_type=jnp.float32) + b_ref[...]

    out_ref[...] = (jnp.dot(h, wo_ref[...], preferred_element_type=jnp.float32)
                    + bo_ref[...])


@jax.jit
def _tc_mlp(x, sums, cnts, w1a, w1b, b1, w2, b2, w3, b3, wo, bo, g, bt):
    return pl.pallas_call(
        _tc_mlp_body,
        out_shape=jax.ShapeDtypeStruct((N_NODES, 64), jnp.float32),
    )(x, sums, cnts, w1a, w1b, b1, w2, b2, w3, b3, wo, bo, g, bt)


def kernel(x, edge_index, edge_attr, W1, b1, W2, b2, W3, b3, Wout, bout,
           gamma, beta):
    src = edge_index[0].astype(jnp.int32)
    src2d = src.reshape(N_EDGES // BATCH, BATCH)
    sums, cnts = _sc_scatter(src2d, edge_attr)
    r = lambda v: v.reshape(1, -1)
    return _tc_mlp(x, sums, cnts, W1[:NODE_DIM], W1[NODE_DIM:], r(b1),
                   W2, r(b2), W3, r(b3), Wout, r(bout), r(gamma), r(beta))


# speedup vs baseline: 6.1391x; 6.1391x over previous
import jax
import jax.numpy as jnp
from jax import lax
from jax.experimental import pallas as pl
from jax.experimental.pallas import tpu as pltpu
from jax.experimental.pallas import tpu_sc as plsc

N_NODES = 10000
N_EDGES = 320000
EDGE_DIM = 16
NODE_DIM = 128

NC = 2
NS = 16
NW = NC * NS
EPW = N_EDGES // NW
BATCH = 100
NB = EPW // BATCH
CHUNK = 5000
N_CHUNKS = EPW // CHUNK
BPC = CHUNK // BATCH
ROWS_PER_TILE = N_NODES // NS


def _sc_scatter_body(src3d_hbm, eattr_hbm, out_sums, out_cnts,
                     ebuf, ibuf, ones_v, zer_v, sums_sh, cnts_sh):
    c = lax.axis_index("c")
    s = lax.axis_index("s")
    w = c * NS + s

    z16 = jnp.zeros((16,), jnp.float32)
    o16 = jnp.ones((16,), jnp.float32)

    def fill_z(i, carry):
        zer_v[i, :] = z16
        return carry

    lax.fori_loop(0, ROWS_PER_TILE, fill_z, 0)

    def fill_o(i, carry):
        ones_v[i, :] = o16
        return carry

    lax.fori_loop(0, BATCH, fill_o, 0)

    pltpu.sync_copy(zer_v, sums_sh.at[pl.ds(s * ROWS_PER_TILE, ROWS_PER_TILE)])
    pltpu.sync_copy(zer_v, cnts_sh.at[pl.ds(s * ROWS_PER_TILE, ROWS_PER_TILE)])

    pltpu.sync_copy(src3d_hbm.at[w], ibuf)
    plsc.subcore_barrier()

    for k in range(N_CHUNKS):
        ebase = w * EPW + k * CHUNK
        pltpu.sync_copy(eattr_hbm.at[pl.ds(ebase, CHUNK)], ebuf)

        def scat(b, carry):
            idx = ibuf.at[k * BPC + b]
            pltpu.sync_copy(ebuf.at[pl.ds(b * BATCH, BATCH)],
                            sums_sh.at[idx], add=True)
            pltpu.sync_copy(ones_v, cnts_sh.at[idx], add=True)
            return carry

        lax.fori_loop(0, BPC, scat, 0)

    plsc.subcore_barrier()

    row0 = s * ROWS_PER_TILE
    pltpu.sync_copy(sums_sh.at[pl.ds(row0, ROWS_PER_TILE)], out_sums.at[c, s])
    pltpu.sync_copy(cnts_sh.at[pl.ds(row0, ROWS_PER_TILE)], out_cnts.at[c, s])


@jax.jit
def _sc_scatter(src3d, edge_attr):
    mesh = plsc.VectorSubcoreMesh(core_axis_name="c", subcore_axis_name="s")
    f = pl.kernel(
        _sc_scatter_body,
        out_type=(
            jax.ShapeDtypeStruct((NC, NS, ROWS_PER_TILE, EDGE_DIM), jnp.float32),
            jax.ShapeDtypeStruct((NC, NS, ROWS_PER_TILE, EDGE_DIM), jnp.float32),
        ),
        mesh=mesh,
        compiler_params=pltpu.CompilerParams(use_tc_tiling_on_sc=False),
        scratch_types=[
            pltpu.VMEM((CHUNK, EDGE_DIM), jnp.float32),
            pltpu.VMEM((NB, BATCH), jnp.int32),
            pltpu.VMEM((BATCH, EDGE_DIM), jnp.float32),
            pltpu.VMEM((ROWS_PER_TILE, EDGE_DIM), jnp.float32),
            pltpu.VMEM_SHARED((N_NODES, EDGE_DIM), jnp.float32),
            pltpu.VMEM_SHARED((N_NODES, EDGE_DIM), jnp.float32),
        ],
    )
    return f(src3d, edge_attr)


def _tc_mlp_body(x_ref, sums_ref, cnts_ref, w1a_ref, w1b_ref, b1_ref,
                 w2_ref, b2_ref, w3_ref, b3_ref, wo_ref, bo_ref,
                 g_ref, bt_ref, out_ref):
    sums = sums_ref[0] + sums_ref[1]
    cnt = cnts_ref[0, :, 0:1] + cnts_ref[1, :, 0:1]
    agg = sums / jnp.maximum(cnt, 1.0)

    g = g_ref[...]
    bt = bt_ref[...]

    h = (jnp.dot(x_ref[...], w1a_ref[...], preferred_element_type=jnp.float32)
         + jnp.dot(agg, w1b_ref[...], preferred_element_type=jnp.float32)
         + b1_ref[...])

    for w_ref, b_ref in ((w2_ref, b2_ref), (w3_ref, b3_ref), (None, None)):
        h = jnp.maximum(h, 0.0)
        mu = jnp.mean(h, axis=0, keepdims=True)
        d = h - mu
        var = jnp.mean(d * d, axis=0, keepdims=True)
        h = g * d / jnp.sqrt(var + 1e-5) + bt
        if w_ref is not None:
            h = jnp.dot(h, w_ref[...], preferred_element_type=jnp.float32) + b_ref[...]

    out_ref[...] = (jnp.dot(h, wo_ref[...], preferred_element_type=jnp.float32)
                    + bo_ref[...])


@jax.jit
def _tc_mlp(x, sums, cnts, w1a, w1b, b1, w2, b2, w3, b3, wo, bo, g, bt):
    return pl.pallas_call(
        _tc_mlp_body,
        out_shape=jax.ShapeDtypeStruct((N_NODES, 64), jnp.float32),
    )(x, sums, cnts, w1a, w1b, b1, w2, b2, w3, b3, wo, bo, g, bt)


def kernel(x, edge_index, edge_attr, W1, b1, W2, b2, W3, b3, Wout, bout,
           gamma, beta):
    src = edge_index[0].astype(jnp.int32)
    src3d = src.reshape(NW, NB, BATCH)
    sums, cnts = _sc_scatter(src3d, edge_attr)
    sums = sums.reshape(NC, N_NODES, EDGE_DIM)
    cnts = cnts.reshape(NC, N_NODES, EDGE_DIM)
    r = lambda v: v.reshape(1, -1)
    return _tc_mlp(x, sums, cnts, W1[:NODE_DIM], W1[NODE_DIM:], r(b1),
                   W2, r(b2), W3, r(b3), Wout, r(bout), r(gamma), r(beta))
